# baseline (speedup 1.0000x reference)
import functools

import jax
import jax.numpy as jnp
from jax import lax
from jax.experimental import pallas as pl
from jax.experimental.pallas import tpu as pltpu
from jax.experimental.pallas import tpu_sc as plsc

N = 65536
PTS = 128
NW = 32
RPW = N // NW
CHUNK = 64
NCHUNK = RPW // CHUNK
GROUPS = CHUNK // 16

W1 = 8.0 / 14.0
W2 = 4.0 / 14.0
W3 = 2.0 / 14.0
BIG = 3.0e38


def _sc_stats_kernel(ps_hbm, out_hbm, xb0, yb0, zb0, xb1, yb1, zb1, obuf,
                     sem0, sem1):
    wid = lax.axis_index("s") * 2 + lax.axis_index("c")
    lane = lax.iota(jnp.int32, 16)
    big = jnp.full((16,), BIG, jnp.float32)
    zero = jnp.zeros((16,), jnp.float32)
    col0 = jnp.zeros((16,), jnp.int32)
    skew = lane * 17

    sets = ((xb0, yb0, zb0, sem0), (xb1, yb1, zb1, sem1))

    def issue(ci, s):
        xbuf, ybuf, zbuf, sem = s
        row0 = ci * CHUNK
        pltpu.async_copy(ps_hbm.at[pl.ds((wid * 3 + 0) * RPW + row0, CHUNK),
                                   :], xbuf, sem)
        pltpu.async_copy(ps_hbm.at[pl.ds((wid * 3 + 1) * RPW + row0, CHUNK),
                                   :], ybuf, sem)
        pltpu.async_copy(ps_hbm.at[pl.ds((wid * 3 + 2) * RPW + row0, CHUNK),
                                   :], zbuf, sem)

    def drain(s):
        xbuf, ybuf, zbuf, sem = s
        dummy = ps_hbm.at[pl.ds(0, CHUNK), :]
        pltpu.make_async_copy(dummy, xbuf, sem).wait()
        pltpu.make_async_copy(dummy, ybuf, sem).wait()
        pltpu.make_async_copy(dummy, zbuf, sem).wait()

    def compute(ci, s):
        xbuf, ybuf, zbuf, _ = s
        for g in range(GROUPS):
            rows = lane + g * 16

            def p_sum(p, c):
                sx, sy, sz, az1, az2, az3, nz1, nz2, nz3 = c
                col = (skew + p) & 127
                x = plsc.load_gather(xbuf, [rows, col])
                y = plsc.load_gather(ybuf, [rows, col])
                z = plsc.load_gather(zbuf, [rows, col])
                l1 = jnp.minimum(az1, z); az1 = jnp.maximum(az1, z)
                l2 = jnp.minimum(az2, l1); az2 = jnp.maximum(az2, l1)
                az3 = jnp.maximum(az3, l2)
                h1 = jnp.maximum(nz1, z); nz1 = jnp.minimum(nz1, z)
                h2 = jnp.maximum(nz2, h1); nz2 = jnp.minimum(nz2, h1)
                nz3 = jnp.minimum(nz3, h2)
                return (sx + x, sy + y, sz + z,
                        az1, az2, az3, nz1, nz2, nz3)

            (sx, sy, sz, az1, az2, az3, nz1, nz2, nz3) = lax.fori_loop(
                0, PTS, p_sum,
                (zero, zero, zero, -big, -big, -big, big, big, big),
                unroll=8)
            inv = 1.0 / PTS
            cx = sx * inv
            cy = sy * inv
            cz = sz * inv
            x0 = plsc.load_gather(xbuf, [rows, col0])
            y0 = plsc.load_gather(ybuf, [rows, col0])
            dx = x0 - cx
            dy = y0 - cy

            def p_topk(p, c):
                (ax1, ax2, ax3, nx1, nx2, nx3,
                 ay1, ay2, ay3, ny1, ny2, ny3) = c
                col = (skew + p) & 127
                x = plsc.load_gather(xbuf, [rows, col])
                y = plsc.load_gather(ybuf, [rows, col])
                px = x * dx + y * dy
                py = x * dy - y * dx
                l1 = jnp.minimum(ax1, px); ax1 = jnp.maximum(ax1, px)
                l2 = jnp.minimum(ax2, l1); ax2 = jnp.maximum(ax2, l1)
                ax3 = jnp.maximum(ax3, l2)
                h1 = jnp.maximum(nx1, px); nx1 = jnp.minimum(nx1, px)
                h2 = jnp.maximum(nx2, h1); nx2 = jnp.minimum(nx2, h1)
                nx3 = jnp.minimum(nx3, h2)

                l1 = jnp.minimum(ay1, py); ay1 = jnp.maximum(ay1, py)
                l2 = jnp.minimum(ay2, l1); ay2 = jnp.maximum(ay2, l1)
                ay3 = jnp.maximum(ay3, l2)
                h1 = jnp.maximum(ny1, py); ny1 = jnp.minimum(ny1, py)
                h2 = jnp.maximum(ny2, h1); ny2 = jnp.minimum(ny2, h1)
                ny3 = jnp.minimum(ny3, h2)
                return (ax1, ax2, ax3, nx1, nx2, nx3,
                        ay1, ay2, ay3, ny1, ny2, ny3)

            init = (-big, -big, -big, big, big, big,
                    -big, -big, -big, big, big, big)
            (ax1, ax2, ax3, nx1, nx2, nx3,
             ay1, ay2, ay3, ny1, ny2, ny3) = lax.fori_loop(
                 0, PTS, p_topk, init, unroll=8)

            ex = W1 * (ax1 - nx1) + W2 * (ax2 - nx2) + W3 * (ax3 - nx3)
            ey = W1 * (ay1 - ny1) + W2 * (ay2 - ny2) + W3 * (ay3 - ny3)
            ez = W1 * (az1 - nz1) + W2 * (az2 - nz2) + W3 * (az3 - nz3)

            off = ci * CHUNK + g * 16
            obuf[pl.ds(0 * RPW + off, 16)] = cx
            obuf[pl.ds(1 * RPW + off, 16)] = cy
            obuf[pl.ds(2 * RPW + off, 16)] = cz
            obuf[pl.ds(3 * RPW + off, 16)] = dx
            obuf[pl.ds(4 * RPW + off, 16)] = dy
            obuf[pl.ds(5 * RPW + off, 16)] = ex
            obuf[pl.ds(6 * RPW + off, 16)] = ey
            obuf[pl.ds(7 * RPW + off, 16)] = ez

    issue(0, sets[0])

    def pair_body(pi, carry):
        ci0 = pi * 2
        issue(ci0 + 1, sets[1])
        drain(sets[0])
        compute(ci0, sets[0])

        @pl.when(pi < NCHUNK // 2 - 1)
        def _():
            issue(ci0 + 2, sets[0])

        drain(sets[1])
        compute(ci0 + 1, sets[1])
        return carry

    lax.fori_loop(0, NCHUNK // 2, pair_body, 0)
    pltpu.sync_copy(obuf, out_hbm.at[pl.ds(wid * 8 * RPW, 8 * RPW)])


_sc_stats = functools.partial(
    pl.kernel,
    mesh=plsc.VectorSubcoreMesh(core_axis_name="c", subcore_axis_name="s"),
    out_type=jax.ShapeDtypeStruct((NW * 8 * RPW,), jnp.float32),
    scratch_types=[
        pltpu.VMEM((CHUNK, PTS), jnp.float32),
        pltpu.VMEM((CHUNK, PTS), jnp.float32),
        pltpu.VMEM((CHUNK, PTS), jnp.float32),
        pltpu.VMEM((CHUNK, PTS), jnp.float32),
        pltpu.VMEM((CHUNK, PTS), jnp.float32),
        pltpu.VMEM((CHUNK, PTS), jnp.float32),
        pltpu.VMEM((8 * RPW,), jnp.float32),
        pltpu.SemaphoreType.DMA,
        pltpu.SemaphoreType.DMA,
    ],
    compiler_params=pltpu.CompilerParams(needs_layout_passes=False),
)(_sc_stats_kernel)


def _tc_post_kernel(in_ref, out_ref):
    v = in_ref[0]
    cx, cy, cz = v[0], v[1], v[2]
    dx, dy = v[3], v[4]
    ex, ey, ez = v[5], v[6], v[7]
    n = jnp.sqrt(dx * dx + dy * dy)
    nc = jnp.maximum(n, jnp.float32(1e-12))
    inv = jnp.float32(1.0) / nc
    sin = dy * inv
    cos = dx * inv
    logn = jnp.log(nc)
    length = jnp.log(ex) - logn
    width = jnp.log(ey) - logn
    height = jnp.log(ez)
    out_ref[...] = jnp.stack(
        [cx, cy, length, width, cz, height, sin, cos], axis=0)[None]


def kernel(point_set):
    planes = jnp.transpose(point_set, (0, 1, 4, 2, 3)).reshape(NW * 3 * RPW,
                                                               PTS)
    stats = _sc_stats(planes)
    stats = stats.reshape(NW, 8, RPW)
    post = pl.pallas_call(
        _tc_post_kernel,
        grid=(NW,),
        in_specs=[pl.BlockSpec((1, 8, RPW), lambda i: (i, 0, 0))],
        out_specs=pl.BlockSpec((1, 8, RPW), lambda i: (i, 0, 0)),
        out_shape=jax.ShapeDtypeStruct((NW, 8, RPW), jnp.float32),
    )(stats)
    return post.reshape(4, 8, 8, RPW).transpose(0, 1, 3, 2)


# speedup vs baseline: 1.0319x; 1.0319x over previous
import functools

import jax
import jax.numpy as jnp
from jax import lax
from jax.experimental import pallas as pl
from jax.experimental.pallas import tpu as pltpu
from jax.experimental.pallas import tpu_sc as plsc

N = 65536
PTS = 128
NW = 32
RPW = N // NW
CHUNK = 128
NCHUNK = RPW // CHUNK
GROUPS = CHUNK // 16

W1 = 8.0 / 14.0
W2 = 4.0 / 14.0
W3 = 2.0 / 14.0
BIG = 3.0e38


def _sc_stats_kernel(ps_hbm, out_hbm, xb0, yb0, zb0, xb1, yb1, zb1, obuf,
                     sem0, sem1):
    wid = lax.axis_index("s") * 2 + lax.axis_index("c")
    lane = lax.iota(jnp.int32, 16)
    big = jnp.full((16,), BIG, jnp.float32)
    zero = jnp.zeros((16,), jnp.float32)
    col0 = jnp.zeros((16,), jnp.int32)
    skew = lane * 17

    sets = ((xb0, yb0, zb0, sem0), (xb1, yb1, zb1, sem1))

    def issue(ci, s):
        xbuf, ybuf, zbuf, sem = s
        row0 = ci * CHUNK
        pltpu.async_copy(ps_hbm.at[pl.ds((wid * 3 + 0) * RPW + row0, CHUNK),
                                   :], xbuf, sem)
        pltpu.async_copy(ps_hbm.at[pl.ds((wid * 3 + 1) * RPW + row0, CHUNK),
                                   :], ybuf, sem)
        pltpu.async_copy(ps_hbm.at[pl.ds((wid * 3 + 2) * RPW + row0, CHUNK),
                                   :], zbuf, sem)

    def drain(s):
        xbuf, ybuf, zbuf, sem = s
        dummy = ps_hbm.at[pl.ds(0, CHUNK), :]
        pltpu.make_async_copy(dummy, xbuf, sem).wait()
        pltpu.make_async_copy(dummy, ybuf, sem).wait()
        pltpu.make_async_copy(dummy, zbuf, sem).wait()

    def compute(ci, s):
        xbuf, ybuf, zbuf, _ = s
        for g in range(GROUPS):
            rows = lane + g * 16

            def p_sum(p, c):
                sx, sy, sz, az1, az2, az3, nz1, nz2, nz3 = c
                col = (skew + p) & 127
                x = plsc.load_gather(xbuf, [rows, col])
                y = plsc.load_gather(ybuf, [rows, col])
                z = plsc.load_gather(zbuf, [rows, col])
                l1 = jnp.minimum(az1, z); az1 = jnp.maximum(az1, z)
                l2 = jnp.minimum(az2, l1); az2 = jnp.maximum(az2, l1)
                az3 = jnp.maximum(az3, l2)
                h1 = jnp.maximum(nz1, z); nz1 = jnp.minimum(nz1, z)
                h2 = jnp.maximum(nz2, h1); nz2 = jnp.minimum(nz2, h1)
                nz3 = jnp.minimum(nz3, h2)
                return (sx + x, sy + y, sz + z,
                        az1, az2, az3, nz1, nz2, nz3)

            (sx, sy, sz, az1, az2, az3, nz1, nz2, nz3) = lax.fori_loop(
                0, PTS, p_sum,
                (zero, zero, zero, -big, -big, -big, big, big, big),
                unroll=8)
            inv = 1.0 / PTS
            cx = sx * inv
            cy = sy * inv
            cz = sz * inv
            x0 = plsc.load_gather(xbuf, [rows, col0])
            y0 = plsc.load_gather(ybuf, [rows, col0])
            dx = x0 - cx
            dy = y0 - cy

            def p_topk(p, c):
                (ax1, ax2, ax3, nx1, nx2, nx3,
                 ay1, ay2, ay3, ny1, ny2, ny3) = c
                col = (skew + p) & 127
                x = plsc.load_gather(xbuf, [rows, col])
                y = plsc.load_gather(ybuf, [rows, col])
                px = x * dx + y * dy
                py = x * dy - y * dx
                l1 = jnp.minimum(ax1, px); ax1 = jnp.maximum(ax1, px)
                l2 = jnp.minimum(ax2, l1); ax2 = jnp.maximum(ax2, l1)
                ax3 = jnp.maximum(ax3, l2)
                h1 = jnp.maximum(nx1, px); nx1 = jnp.minimum(nx1, px)
                h2 = jnp.maximum(nx2, h1); nx2 = jnp.minimum(nx2, h1)
                nx3 = jnp.minimum(nx3, h2)

                l1 = jnp.minimum(ay1, py); ay1 = jnp.maximum(ay1, py)
                l2 = jnp.minimum(ay2, l1); ay2 = jnp.maximum(ay2, l1)
                ay3 = jnp.maximum(ay3, l2)
                h1 = jnp.maximum(ny1, py); ny1 = jnp.minimum(ny1, py)
                h2 = jnp.maximum(ny2, h1); ny2 = jnp.minimum(ny2, h1)
                ny3 = jnp.minimum(ny3, h2)
                return (ax1, ax2, ax3, nx1, nx2, nx3,
                        ay1, ay2, ay3, ny1, ny2, ny3)

            init = (-big, -big, -big, big, big, big,
                    -big, -big, -big, big, big, big)
            (ax1, ax2, ax3, nx1, nx2, nx3,
             ay1, ay2, ay3, ny1, ny2, ny3) = lax.fori_loop(
                 0, PTS, p_topk, init, unroll=4)

            ex = W1 * (ax1 - nx1) + W2 * (ax2 - nx2) + W3 * (ax3 - nx3)
            ey = W1 * (ay1 - ny1) + W2 * (ay2 - ny2) + W3 * (ay3 - ny3)
            ez = W1 * (az1 - nz1) + W2 * (az2 - nz2) + W3 * (az3 - nz3)

            off = ci * CHUNK + g * 16
            obuf[pl.ds(0 * RPW + off, 16)] = cx
            obuf[pl.ds(1 * RPW + off, 16)] = cy
            obuf[pl.ds(2 * RPW + off, 16)] = cz
            obuf[pl.ds(3 * RPW + off, 16)] = dx
            obuf[pl.ds(4 * RPW + off, 16)] = dy
            obuf[pl.ds(5 * RPW + off, 16)] = ex
            obuf[pl.ds(6 * RPW + off, 16)] = ey
            obuf[pl.ds(7 * RPW + off, 16)] = ez

    issue(0, sets[0])

    def pair_body(pi, carry):
        ci0 = pi * 2
        issue(ci0 + 1, sets[1])
        drain(sets[0])
        compute(ci0, sets[0])

        @pl.when(pi < NCHUNK // 2 - 1)
        def _():
            issue(ci0 + 2, sets[0])

        drain(sets[1])
        compute(ci0 + 1, sets[1])
        return carry

    lax.fori_loop(0, NCHUNK // 2, pair_body, 0)
    pltpu.sync_copy(obuf, out_hbm.at[pl.ds(wid * 8 * RPW, 8 * RPW)])


_sc_stats = functools.partial(
    pl.kernel,
    mesh=plsc.VectorSubcoreMesh(core_axis_name="c", subcore_axis_name="s"),
    out_type=jax.ShapeDtypeStruct((NW * 8 * RPW,), jnp.float32),
    scratch_types=[
        pltpu.VMEM((CHUNK, PTS), jnp.float32),
        pltpu.VMEM((CHUNK, PTS), jnp.float32),
        pltpu.VMEM((CHUNK, PTS), jnp.float32),
        pltpu.VMEM((CHUNK, PTS), jnp.float32),
        pltpu.VMEM((CHUNK, PTS), jnp.float32),
        pltpu.VMEM((CHUNK, PTS), jnp.float32),
        pltpu.VMEM((8 * RPW,), jnp.float32),
        pltpu.SemaphoreType.DMA,
        pltpu.SemaphoreType.DMA,
    ],
    compiler_params=pltpu.CompilerParams(needs_layout_passes=False),
)(_sc_stats_kernel)


def _tc_post_kernel(in_ref, out_ref):
    v = in_ref[0]
    cx, cy, cz = v[0], v[1], v[2]
    dx, dy = v[3], v[4]
    ex, ey, ez = v[5], v[6], v[7]
    n = jnp.sqrt(dx * dx + dy * dy)
    nc = jnp.maximum(n, jnp.float32(1e-12))
    inv = jnp.float32(1.0) / nc
    sin = dy * inv
    cos = dx * inv
    logn = jnp.log(nc)
    length = jnp.log(ex) - logn
    width = jnp.log(ey) - logn
    height = jnp.log(ez)
    out_ref[...] = jnp.stack(
        [cx, cy, length, width, cz, height, sin, cos], axis=0)[None]


def kernel(point_set):
    planes = jnp.transpose(point_set, (0, 1, 4, 2, 3)).reshape(NW * 3 * RPW,
                                                               PTS)
    stats = _sc_stats(planes)
    stats = stats.reshape(NW, 8, RPW)
    post = pl.pallas_call(
        _tc_post_kernel,
        grid=(NW,),
        in_specs=[pl.BlockSpec((1, 8, RPW), lambda i: (i, 0, 0))],
        out_specs=pl.BlockSpec((1, 8, RPW), lambda i: (i, 0, 0)),
        out_shape=jax.ShapeDtypeStruct((NW, 8, RPW), jnp.float32),
    )(stats)
    return post.reshape(4, 8, 8, RPW).transpose(0, 1, 3, 2)


# speedup vs baseline: 1.0836x; 1.0501x over previous
import functools

import jax
import jax.numpy as jnp
from jax import lax
from jax.experimental import pallas as pl
from jax.experimental.pallas import tpu as pltpu
from jax.experimental.pallas import tpu_sc as plsc

N = 65536
PTS = 128
NW = 32
RPW = N // NW
CHUNK = 32
NCHUNK = RPW // CHUNK
GROUPS = CHUNK // 16

W1 = 8.0 / 14.0
W2 = 4.0 / 14.0
W3 = 2.0 / 14.0
BIG = 3.0e38


def _sc_stats_kernel(ps_hbm, out_hbm, xb0, yb0, zb0, xb1, yb1, zb1, obuf,
                     sem0, sem1):
    wid = lax.axis_index("s") * 2 + lax.axis_index("c")
    lane = lax.iota(jnp.int32, 16)
    big = jnp.full((16,), BIG, jnp.float32)
    zero = jnp.zeros((16,), jnp.float32)
    col0 = jnp.zeros((16,), jnp.int32)
    skew = lane * 17

    sets = ((xb0, yb0, zb0, sem0), (xb1, yb1, zb1, sem1))

    def issue(ci, s):
        xbuf, ybuf, zbuf, sem = s
        row0 = ci * CHUNK
        pltpu.async_copy(ps_hbm.at[pl.ds((wid * 3 + 0) * RPW + row0, CHUNK),
                                   :], xbuf, sem)
        pltpu.async_copy(ps_hbm.at[pl.ds((wid * 3 + 1) * RPW + row0, CHUNK),
                                   :], ybuf, sem)
        pltpu.async_copy(ps_hbm.at[pl.ds((wid * 3 + 2) * RPW + row0, CHUNK),
                                   :], zbuf, sem)

    def drain(s):
        xbuf, ybuf, zbuf, sem = s
        dummy = ps_hbm.at[pl.ds(0, CHUNK), :]
        pltpu.make_async_copy(dummy, xbuf, sem).wait()
        pltpu.make_async_copy(dummy, ybuf, sem).wait()
        pltpu.make_async_copy(dummy, zbuf, sem).wait()

    def compute(ci, s):
        xbuf, ybuf, zbuf, _ = s
        for g in range(GROUPS):
            rows = lane + g * 16

            def p_sum(p, c):
                sx, sy, sz, az1, az2, az3, nz1, nz2, nz3 = c
                col = (skew + p) & 127
                x = plsc.load_gather(xbuf, [rows, col])
                y = plsc.load_gather(ybuf, [rows, col])
                z = plsc.load_gather(zbuf, [rows, col])
                l1 = jnp.minimum(az1, z); az1 = jnp.maximum(az1, z)
                l2 = jnp.minimum(az2, l1); az2 = jnp.maximum(az2, l1)
                az3 = jnp.maximum(az3, l2)
                h1 = jnp.maximum(nz1, z); nz1 = jnp.minimum(nz1, z)
                h2 = jnp.maximum(nz2, h1); nz2 = jnp.minimum(nz2, h1)
                nz3 = jnp.minimum(nz3, h2)
                return (sx + x, sy + y, sz + z,
                        az1, az2, az3, nz1, nz2, nz3)

            (sx, sy, sz, az1, az2, az3, nz1, nz2, nz3) = lax.fori_loop(
                0, PTS, p_sum,
                (zero, zero, zero, -big, -big, -big, big, big, big),
                unroll=8)
            inv = 1.0 / PTS
            cx = sx * inv
            cy = sy * inv
            cz = sz * inv
            x0 = plsc.load_gather(xbuf, [rows, col0])
            y0 = plsc.load_gather(ybuf, [rows, col0])
            dx = x0 - cx
            dy = y0 - cy

            def p_topk(p, c):
                (ax1, ax2, ax3, nx1, nx2, nx3,
                 ay1, ay2, ay3, ny1, ny2, ny3) = c
                col = (skew + p) & 127
                x = plsc.load_gather(xbuf, [rows, col])
                y = plsc.load_gather(ybuf, [rows, col])
                px = x * dx + y * dy
                py = x * dy - y * dx
                l1 = jnp.minimum(ax1, px); ax1 = jnp.maximum(ax1, px)
                l2 = jnp.minimum(ax2, l1); ax2 = jnp.maximum(ax2, l1)
                ax3 = jnp.maximum(ax3, l2)
                h1 = jnp.maximum(nx1, px); nx1 = jnp.minimum(nx1, px)
                h2 = jnp.maximum(nx2, h1); nx2 = jnp.minimum(nx2, h1)
                nx3 = jnp.minimum(nx3, h2)

                l1 = jnp.minimum(ay1, py); ay1 = jnp.maximum(ay1, py)
                l2 = jnp.minimum(ay2, l1); ay2 = jnp.maximum(ay2, l1)
                ay3 = jnp.maximum(ay3, l2)
                h1 = jnp.maximum(ny1, py); ny1 = jnp.minimum(ny1, py)
                h2 = jnp.maximum(ny2, h1); ny2 = jnp.minimum(ny2, h1)
                ny3 = jnp.minimum(ny3, h2)
                return (ax1, ax2, ax3, nx1, nx2, nx3,
                        ay1, ay2, ay3, ny1, ny2, ny3)

            init = (-big, -big, -big, big, big, big,
                    -big, -big, -big, big, big, big)
            (ax1, ax2, ax3, nx1, nx2, nx3,
             ay1, ay2, ay3, ny1, ny2, ny3) = lax.fori_loop(
                 0, PTS, p_topk, init, unroll=4)

            ex = W1 * (ax1 - nx1) + W2 * (ax2 - nx2) + W3 * (ax3 - nx3)
            ey = W1 * (ay1 - ny1) + W2 * (ay2 - ny2) + W3 * (ay3 - ny3)
            ez = W1 * (az1 - nz1) + W2 * (az2 - nz2) + W3 * (az3 - nz3)

            off = ci * CHUNK + g * 16
            obuf[pl.ds(0 * RPW + off, 16)] = cx
            obuf[pl.ds(1 * RPW + off, 16)] = cy
            obuf[pl.ds(2 * RPW + off, 16)] = cz
            obuf[pl.ds(3 * RPW + off, 16)] = dx
            obuf[pl.ds(4 * RPW + off, 16)] = dy
            obuf[pl.ds(5 * RPW + off, 16)] = ex
            obuf[pl.ds(6 * RPW + off, 16)] = ey
            obuf[pl.ds(7 * RPW + off, 16)] = ez

    issue(0, sets[0])

    def pair_body(pi, carry):
        ci0 = pi * 2
        issue(ci0 + 1, sets[1])
        drain(sets[0])
        compute(ci0, sets[0])

        @pl.when(pi < NCHUNK // 2 - 1)
        def _():
            issue(ci0 + 2, sets[0])

        drain(sets[1])
        compute(ci0 + 1, sets[1])
        return carry

    lax.fori_loop(0, NCHUNK // 2, pair_body, 0)
    pltpu.sync_copy(obuf, out_hbm.at[pl.ds(wid * 8 * RPW, 8 * RPW)])


_sc_stats = functools.partial(
    pl.kernel,
    mesh=plsc.VectorSubcoreMesh(core_axis_name="c", subcore_axis_name="s"),
    out_type=jax.ShapeDtypeStruct((NW * 8 * RPW,), jnp.float32),
    scratch_types=[
        pltpu.VMEM((CHUNK, PTS), jnp.float32),
        pltpu.VMEM((CHUNK, PTS), jnp.float32),
        pltpu.VMEM((CHUNK, PTS), jnp.float32),
        pltpu.VMEM((CHUNK, PTS), jnp.float32),
        pltpu.VMEM((CHUNK, PTS), jnp.float32),
        pltpu.VMEM((CHUNK, PTS), jnp.float32),
        pltpu.VMEM((8 * RPW,), jnp.float32),
        pltpu.SemaphoreType.DMA,
        pltpu.SemaphoreType.DMA,
    ],
    compiler_params=pltpu.CompilerParams(needs_layout_passes=False),
)(_sc_stats_kernel)


def _tc_post_kernel(in_ref, out_ref):
    v = in_ref[0]
    cx, cy, cz = v[0], v[1], v[2]
    dx, dy = v[3], v[4]
    ex, ey, ez = v[5], v[6], v[7]
    n = jnp.sqrt(dx * dx + dy * dy)
    nc = jnp.maximum(n, jnp.float32(1e-12))
    inv = jnp.float32(1.0) / nc
    sin = dy * inv
    cos = dx * inv
    logn = jnp.log(nc)
    length = jnp.log(ex) - logn
    width = jnp.log(ey) - logn
    height = jnp.log(ez)
    out_ref[...] = jnp.stack(
        [cx, cy, length, width, cz, height, sin, cos], axis=0)[None]


def kernel(point_set):
    planes = jnp.transpose(point_set, (0, 1, 4, 2, 3)).reshape(NW * 3 * RPW,
                                                               PTS)
    stats = _sc_stats(planes)
    stats = stats.reshape(NW, 8, RPW)
    post = pl.pallas_call(
        _tc_post_kernel,
        grid=(NW,),
        in_specs=[pl.BlockSpec((1, 8, RPW), lambda i: (i, 0, 0))],
        out_specs=pl.BlockSpec((1, 8, RPW), lambda i: (i, 0, 0)),
        out_shape=jax.ShapeDtypeStruct((NW, 8, RPW), jnp.float32),
    )(stats)
    return post.reshape(4, 8, 8, RPW).transpose(0, 1, 3, 2)


# speedup vs baseline: 1.0843x; 1.0007x over previous
import functools

import jax
import jax.numpy as jnp
from jax import lax
from jax.experimental import pallas as pl
from jax.experimental.pallas import tpu as pltpu
from jax.experimental.pallas import tpu_sc as plsc

N = 65536
PTS = 128
NW = 32
RPW = N // NW
CHUNK = 16
NCHUNK = RPW // CHUNK
GROUPS = CHUNK // 16

W1 = 8.0 / 14.0
W2 = 4.0 / 14.0
W3 = 2.0 / 14.0
BIG = 3.0e38


def _sc_stats_kernel(ps_hbm, out_hbm, xb0, yb0, zb0, xb1, yb1, zb1, obuf,
                     sem0, sem1):
    wid = lax.axis_index("s") * 2 + lax.axis_index("c")
    lane = lax.iota(jnp.int32, 16)
    big = jnp.full((16,), BIG, jnp.float32)
    zero = jnp.zeros((16,), jnp.float32)
    col0 = jnp.zeros((16,), jnp.int32)
    skew = lane * 17

    sets = ((xb0, yb0, zb0, sem0), (xb1, yb1, zb1, sem1))

    def issue(ci, s):
        xbuf, ybuf, zbuf, sem = s
        row0 = ci * CHUNK
        pltpu.async_copy(ps_hbm.at[pl.ds((wid * 3 + 0) * RPW + row0, CHUNK),
                                   :], xbuf, sem)
        pltpu.async_copy(ps_hbm.at[pl.ds((wid * 3 + 1) * RPW + row0, CHUNK),
                                   :], ybuf, sem)
        pltpu.async_copy(ps_hbm.at[pl.ds((wid * 3 + 2) * RPW + row0, CHUNK),
                                   :], zbuf, sem)

    def drain(s):
        xbuf, ybuf, zbuf, sem = s
        dummy = ps_hbm.at[pl.ds(0, CHUNK), :]
        pltpu.make_async_copy(dummy, xbuf, sem).wait()
        pltpu.make_async_copy(dummy, ybuf, sem).wait()
        pltpu.make_async_copy(dummy, zbuf, sem).wait()

    def compute(ci, s):
        xbuf, ybuf, zbuf, _ = s
        for g in range(GROUPS):
            rows = lane + g * 16

            def p_sum(p, c):
                sx, sy, sz, az1, az2, az3, nz1, nz2, nz3 = c
                col = (skew + p) & 127
                x = plsc.load_gather(xbuf, [rows, col])
                y = plsc.load_gather(ybuf, [rows, col])
                z = plsc.load_gather(zbuf, [rows, col])
                l1 = jnp.minimum(az1, z); az1 = jnp.maximum(az1, z)
                l2 = jnp.minimum(az2, l1); az2 = jnp.maximum(az2, l1)
                az3 = jnp.maximum(az3, l2)
                h1 = jnp.maximum(nz1, z); nz1 = jnp.minimum(nz1, z)
                h2 = jnp.maximum(nz2, h1); nz2 = jnp.minimum(nz2, h1)
                nz3 = jnp.minimum(nz3, h2)
                return (sx + x, sy + y, sz + z,
                        az1, az2, az3, nz1, nz2, nz3)

            (sx, sy, sz, az1, az2, az3, nz1, nz2, nz3) = lax.fori_loop(
                0, PTS, p_sum,
                (zero, zero, zero, -big, -big, -big, big, big, big),
                unroll=8)
            inv = 1.0 / PTS
            cx = sx * inv
            cy = sy * inv
            cz = sz * inv
            x0 = plsc.load_gather(xbuf, [rows, col0])
            y0 = plsc.load_gather(ybuf, [rows, col0])
            dx = x0 - cx
            dy = y0 - cy

            def p_topk(p, c):
                (ax1, ax2, ax3, nx1, nx2, nx3,
                 ay1, ay2, ay3, ny1, ny2, ny3) = c
                col = (skew + p) & 127
                x = plsc.load_gather(xbuf, [rows, col])
                y = plsc.load_gather(ybuf, [rows, col])
                px = x * dx + y * dy
                py = x * dy - y * dx
                l1 = jnp.minimum(ax1, px); ax1 = jnp.maximum(ax1, px)
                l2 = jnp.minimum(ax2, l1); ax2 = jnp.maximum(ax2, l1)
                ax3 = jnp.maximum(ax3, l2)
                h1 = jnp.maximum(nx1, px); nx1 = jnp.minimum(nx1, px)
                h2 = jnp.maximum(nx2, h1); nx2 = jnp.minimum(nx2, h1)
                nx3 = jnp.minimum(nx3, h2)

                l1 = jnp.minimum(ay1, py); ay1 = jnp.maximum(ay1, py)
                l2 = jnp.minimum(ay2, l1); ay2 = jnp.maximum(ay2, l1)
                ay3 = jnp.maximum(ay3, l2)
                h1 = jnp.maximum(ny1, py); ny1 = jnp.minimum(ny1, py)
                h2 = jnp.maximum(ny2, h1); ny2 = jnp.minimum(ny2, h1)
                ny3 = jnp.minimum(ny3, h2)
                return (ax1, ax2, ax3, nx1, nx2, nx3,
                        ay1, ay2, ay3, ny1, ny2, ny3)

            init = (-big, -big, -big, big, big, big,
                    -big, -big, -big, big, big, big)
            (ax1, ax2, ax3, nx1, nx2, nx3,
             ay1, ay2, ay3, ny1, ny2, ny3) = lax.fori_loop(
                 0, PTS, p_topk, init, unroll=4)

            ex = W1 * (ax1 - nx1) + W2 * (ax2 - nx2) + W3 * (ax3 - nx3)
            ey = W1 * (ay1 - ny1) + W2 * (ay2 - ny2) + W3 * (ay3 - ny3)
            ez = W1 * (az1 - nz1) + W2 * (az2 - nz2) + W3 * (az3 - nz3)

            off = ci * CHUNK + g * 16
            obuf[pl.ds(0 * RPW + off, 16)] = cx
            obuf[pl.ds(1 * RPW + off, 16)] = cy
            obuf[pl.ds(2 * RPW + off, 16)] = cz
            obuf[pl.ds(3 * RPW + off, 16)] = dx
            obuf[pl.ds(4 * RPW + off, 16)] = dy
            obuf[pl.ds(5 * RPW + off, 16)] = ex
            obuf[pl.ds(6 * RPW + off, 16)] = ey
            obuf[pl.ds(7 * RPW + off, 16)] = ez

    issue(0, sets[0])

    def pair_body(pi, carry):
        ci0 = pi * 2
        issue(ci0 + 1, sets[1])
        drain(sets[0])
        compute(ci0, sets[0])

        @pl.when(pi < NCHUNK // 2 - 1)
        def _():
            issue(ci0 + 2, sets[0])

        drain(sets[1])
        compute(ci0 + 1, sets[1])
        return carry

    lax.fori_loop(0, NCHUNK // 2, pair_body, 0)
    pltpu.sync_copy(obuf, out_hbm.at[pl.ds(wid * 8 * RPW, 8 * RPW)])


_sc_stats = functools.partial(
    pl.kernel,
    mesh=plsc.VectorSubcoreMesh(core_axis_name="c", subcore_axis_name="s"),
    out_type=jax.ShapeDtypeStruct((NW * 8 * RPW,), jnp.float32),
    scratch_types=[
        pltpu.VMEM((CHUNK, PTS), jnp.float32),
        pltpu.VMEM((CHUNK, PTS), jnp.float32),
        pltpu.VMEM((CHUNK, PTS), jnp.float32),
        pltpu.VMEM((CHUNK, PTS), jnp.float32),
        pltpu.VMEM((CHUNK, PTS), jnp.float32),
        pltpu.VMEM((CHUNK, PTS), jnp.float32),
        pltpu.VMEM((8 * RPW,), jnp.float32),
        pltpu.SemaphoreType.DMA,
        pltpu.SemaphoreType.DMA,
    ],
    compiler_params=pltpu.CompilerParams(needs_layout_passes=False),
)(_sc_stats_kernel)


def _tc_post_kernel(in_ref, out_ref):
    v = in_ref[0]
    cx, cy, cz = v[0], v[1], v[2]
    dx, dy = v[3], v[4]
    ex, ey, ez = v[5], v[6], v[7]
    n = jnp.sqrt(dx * dx + dy * dy)
    nc = jnp.maximum(n, jnp.float32(1e-12))
    inv = jnp.float32(1.0) / nc
    sin = dy * inv
    cos = dx * inv
    logn = jnp.log(nc)
    length = jnp.log(ex) - logn
    width = jnp.log(ey) - logn
    height = jnp.log(ez)
    out_ref[...] = jnp.stack(
        [cx, cy, length, width, cz, height, sin, cos], axis=0)[None]


def kernel(point_set):
    planes = jnp.transpose(point_set, (0, 1, 4, 2, 3)).reshape(NW * 3 * RPW,
                                                               PTS)
    stats = _sc_stats(planes)
    stats = stats.reshape(NW, 8, RPW)
    post = pl.pallas_call(
        _tc_post_kernel,
        grid=(NW,),
        in_specs=[pl.BlockSpec((1, 8, RPW), lambda i: (i, 0, 0))],
        out_specs=pl.BlockSpec((1, 8, RPW), lambda i: (i, 0, 0)),
        out_shape=jax.ShapeDtypeStruct((NW, 8, RPW), jnp.float32),
    )(stats)
    return post.reshape(4, 8, 8, RPW).transpose(0, 1, 3, 2)


# speedup vs baseline: 1.1609x; 1.0706x over previous
import functools

import jax
import jax.numpy as jnp
from jax import lax
from jax.experimental import pallas as pl
from jax.experimental.pallas import tpu as pltpu
from jax.experimental.pallas import tpu_sc as plsc

N = 65536
PTS = 128
NW = 32
RPW = N // NW
CHUNK = 16
NCHUNK = RPW // CHUNK
GROUPS = CHUNK // 16

W1 = 8.0 / 14.0
W2 = 4.0 / 14.0
W3 = 2.0 / 14.0
BIG = 3.0e38


def _sc_stats_kernel(ps_hbm, out_hbm, xb0, yb0, zb0, xb1, yb1, zb1, obuf,
                     sem0, sem1):
    wid = lax.axis_index("s") * 2 + lax.axis_index("c")
    lane = lax.iota(jnp.int32, 16)
    big = jnp.full((16,), BIG, jnp.float32)
    zero = jnp.zeros((16,), jnp.float32)
    col0 = jnp.zeros((16,), jnp.int32)
    skew = lane * 17

    sets = ((xb0, yb0, zb0, sem0), (xb1, yb1, zb1, sem1))

    def issue(ci, s):
        xbuf, ybuf, zbuf, sem = s
        row0 = ci * CHUNK
        pltpu.async_copy(ps_hbm.at[pl.ds((wid * 3 + 0) * RPW + row0, CHUNK),
                                   :], xbuf, sem)
        pltpu.async_copy(ps_hbm.at[pl.ds((wid * 3 + 1) * RPW + row0, CHUNK),
                                   :], ybuf, sem)
        pltpu.async_copy(ps_hbm.at[pl.ds((wid * 3 + 2) * RPW + row0, CHUNK),
                                   :], zbuf, sem)

    def drain(s):
        xbuf, ybuf, zbuf, sem = s
        dummy = ps_hbm.at[pl.ds(0, CHUNK), :]
        pltpu.make_async_copy(dummy, xbuf, sem).wait()
        pltpu.make_async_copy(dummy, ybuf, sem).wait()
        pltpu.make_async_copy(dummy, zbuf, sem).wait()

    def compute(ci, s):
        xbuf, ybuf, zbuf, _ = s
        for g in range(GROUPS):
            rows = lane + g * 16

            def merge3_max(a1, a2, a3, hi, lo):
                a1n = jnp.maximum(a1, hi); c1 = jnp.minimum(a1, hi)
                s2 = jnp.maximum(a2, lo); a2n = jnp.maximum(c1, s2)
                s2b = jnp.minimum(a2, lo); t = jnp.minimum(c1, s2)
                a3n = jnp.maximum(t, jnp.maximum(s2b, a3))
                return a1n, a2n, a3n

            def merge3_min(b1, b2, b3, hi, lo):
                b1n = jnp.minimum(b1, lo); c1 = jnp.maximum(b1, lo)
                s2 = jnp.minimum(b2, hi); b2n = jnp.minimum(c1, s2)
                s2b = jnp.maximum(b2, hi); t = jnp.maximum(c1, s2)
                b3n = jnp.minimum(t, jnp.minimum(s2b, b3))
                return b1n, b2n, b3n

            def p_sum(q, c):
                sx, sy, sz, az1, az2, az3, nz1, nz2, nz3 = c
                ca = (skew + 2 * q) & 127
                cb = (skew + 2 * q + 1) & 127
                xa = plsc.load_gather(xbuf, [rows, ca])
                ya = plsc.load_gather(ybuf, [rows, ca])
                za = plsc.load_gather(zbuf, [rows, ca])
                xb = plsc.load_gather(xbuf, [rows, cb])
                yb = plsc.load_gather(ybuf, [rows, cb])
                zb = plsc.load_gather(zbuf, [rows, cb])
                hi = jnp.maximum(za, zb); lo = jnp.minimum(za, zb)
                az1, az2, az3 = merge3_max(az1, az2, az3, hi, lo)
                nz1, nz2, nz3 = merge3_min(nz1, nz2, nz3, hi, lo)
                return (sx + (xa + xb), sy + (ya + yb), sz + (za + zb),
                        az1, az2, az3, nz1, nz2, nz3)

            (sx, sy, sz, az1, az2, az3, nz1, nz2, nz3) = lax.fori_loop(
                0, PTS // 2, p_sum,
                (zero, zero, zero, -big, -big, -big, big, big, big),
                unroll=4)
            inv = 1.0 / PTS
            cx = sx * inv
            cy = sy * inv
            cz = sz * inv
            x0 = plsc.load_gather(xbuf, [rows, col0])
            y0 = plsc.load_gather(ybuf, [rows, col0])
            dx = x0 - cx
            dy = y0 - cy

            def p_topk(q, c):
                (ax1, ax2, ax3, nx1, nx2, nx3,
                 ay1, ay2, ay3, ny1, ny2, ny3) = c
                ca = (skew + 2 * q) & 127
                cb = (skew + 2 * q + 1) & 127
                xa = plsc.load_gather(xbuf, [rows, ca])
                ya = plsc.load_gather(ybuf, [rows, ca])
                xb = plsc.load_gather(xbuf, [rows, cb])
                yb = plsc.load_gather(ybuf, [rows, cb])
                pxa = xa * dx + ya * dy
                pya = xa * dy - ya * dx
                pxb = xb * dx + yb * dy
                pyb = xb * dy - yb * dx
                hi = jnp.maximum(pxa, pxb); lo = jnp.minimum(pxa, pxb)
                ax1, ax2, ax3 = merge3_max(ax1, ax2, ax3, hi, lo)
                nx1, nx2, nx3 = merge3_min(nx1, nx2, nx3, hi, lo)
                hi = jnp.maximum(pya, pyb); lo = jnp.minimum(pya, pyb)
                ay1, ay2, ay3 = merge3_max(ay1, ay2, ay3, hi, lo)
                ny1, ny2, ny3 = merge3_min(ny1, ny2, ny3, hi, lo)
                return (ax1, ax2, ax3, nx1, nx2, nx3,
                        ay1, ay2, ay3, ny1, ny2, ny3)

            init = (-big, -big, -big, big, big, big,
                    -big, -big, -big, big, big, big)
            (ax1, ax2, ax3, nx1, nx2, nx3,
             ay1, ay2, ay3, ny1, ny2, ny3) = lax.fori_loop(
                 0, PTS // 2, p_topk, init, unroll=2)

            ex = W1 * (ax1 - nx1) + W2 * (ax2 - nx2) + W3 * (ax3 - nx3)
            ey = W1 * (ay1 - ny1) + W2 * (ay2 - ny2) + W3 * (ay3 - ny3)
            ez = W1 * (az1 - nz1) + W2 * (az2 - nz2) + W3 * (az3 - nz3)

            off = ci * CHUNK + g * 16
            obuf[pl.ds(0 * RPW + off, 16)] = cx
            obuf[pl.ds(1 * RPW + off, 16)] = cy
            obuf[pl.ds(2 * RPW + off, 16)] = cz
            obuf[pl.ds(3 * RPW + off, 16)] = dx
            obuf[pl.ds(4 * RPW + off, 16)] = dy
            obuf[pl.ds(5 * RPW + off, 16)] = ex
            obuf[pl.ds(6 * RPW + off, 16)] = ey
            obuf[pl.ds(7 * RPW + off, 16)] = ez

    issue(0, sets[0])

    def pair_body(pi, carry):
        ci0 = pi * 2
        issue(ci0 + 1, sets[1])
        drain(sets[0])
        compute(ci0, sets[0])

        @pl.when(pi < NCHUNK // 2 - 1)
        def _():
            issue(ci0 + 2, sets[0])

        drain(sets[1])
        compute(ci0 + 1, sets[1])
        return carry

    lax.fori_loop(0, NCHUNK // 2, pair_body, 0)
    pltpu.sync_copy(obuf, out_hbm.at[pl.ds(wid * 8 * RPW, 8 * RPW)])


_sc_stats = functools.partial(
    pl.kernel,
    mesh=plsc.VectorSubcoreMesh(core_axis_name="c", subcore_axis_name="s"),
    out_type=jax.ShapeDtypeStruct((NW * 8 * RPW,), jnp.float32),
    scratch_types=[
        pltpu.VMEM((CHUNK, PTS), jnp.float32),
        pltpu.VMEM((CHUNK, PTS), jnp.float32),
        pltpu.VMEM((CHUNK, PTS), jnp.float32),
        pltpu.VMEM((CHUNK, PTS), jnp.float32),
        pltpu.VMEM((CHUNK, PTS), jnp.float32),
        pltpu.VMEM((CHUNK, PTS), jnp.float32),
        pltpu.VMEM((8 * RPW,), jnp.float32),
        pltpu.SemaphoreType.DMA,
        pltpu.SemaphoreType.DMA,
    ],
    compiler_params=pltpu.CompilerParams(needs_layout_passes=False),
)(_sc_stats_kernel)


def _tc_post_kernel(in_ref, out_ref):
    v = in_ref[0]
    cx, cy, cz = v[0], v[1], v[2]
    dx, dy = v[3], v[4]
    ex, ey, ez = v[5], v[6], v[7]
    n = jnp.sqrt(dx * dx + dy * dy)
    nc = jnp.maximum(n, jnp.float32(1e-12))
    inv = jnp.float32(1.0) / nc
    sin = dy * inv
    cos = dx * inv
    logn = jnp.log(nc)
    length = jnp.log(ex) - logn
    width = jnp.log(ey) - logn
    height = jnp.log(ez)
    out_ref[...] = jnp.stack(
        [cx, cy, length, width, cz, height, sin, cos], axis=0)[None]


def kernel(point_set):
    planes = jnp.transpose(point_set, (0, 1, 4, 2, 3)).reshape(NW * 3 * RPW,
                                                               PTS)
    stats = _sc_stats(planes)
    stats = stats.reshape(NW, 8, RPW)
    post = pl.pallas_call(
        _tc_post_kernel,
        grid=(NW,),
        in_specs=[pl.BlockSpec((1, 8, RPW), lambda i: (i, 0, 0))],
        out_specs=pl.BlockSpec((1, 8, RPW), lambda i: (i, 0, 0)),
        out_shape=jax.ShapeDtypeStruct((NW, 8, RPW), jnp.float32),
    )(stats)
    return post.reshape(4, 8, 8, RPW).transpose(0, 1, 3, 2)
